# baseline (speedup 1.0000x reference)
import jax
import jax.numpy as jnp
from jax.experimental import pallas as pl

_B = 128
_G = 256
_D = 32
_H = 4
_HD = _D // _H
_E = _H * _G
_BPS = 16


def _one_batch(x_col, z_col, w_row, bemb_row, WqT, WkT, bq, bk, Wg, bg_col,
               W0, b0_row, W1, b1_row, row_io, eye, ones_col):
    f32 = jnp.float32
    X_emb = x_col * w_row + bemb_row
    q = jnp.dot(X_emb, WqT) + bq
    k = jnp.dot(X_emb, WkT) + bk

    Ms = []
    for h in range(_H):
        qh = q[:, h * _HD:(h + 1) * _HD]
        kh = k[:, h * _HD:(h + 1) * _HD]
        st = jax.lax.dot_general(kh, qh, (((1,), (1,)), ((), ())))
        y = st
        for _ in range(5):
            cur = jnp.max(y, axis=0, keepdims=True)
            idx_of = jnp.where(y == cur, row_io, _G)
            jmin = jnp.min(idx_of, axis=0, keepdims=True)
            y = jnp.where(row_io == jmin, -jnp.inf, y)
        Ms.append(jnp.maximum((y == -jnp.inf).astype(f32), eye))

    oms = []
    Dv = jnp.zeros((_G, 1), f32)
    for h in range(_H):
        g_col = jax.nn.sigmoid(jnp.dot(Wg[h * _G:(h + 1) * _G, :], z_col)
                               + bg_col[h * _G:(h + 1) * _G, :])
        nnz = jax.lax.dot_general(Ms[h], ones_col,
                                  (((0,), (0,)), ((), ())))
        de = 1.0 / (g_col * nnz + 1e-8)
        oms.append(g_col * g_col * de)
        Dv = Dv + jnp.dot(Ms[h], g_col)
    dv = jax.lax.rsqrt(Dv + 1e-8)

    X = X_emb
    for layer in range(2):
        Xs = dv * X
        Xa = jnp.zeros((_G, _D), f32)
        for h in range(_H):
            Ye = jax.lax.dot_general(Ms[h], Xs,
                                     (((0,), (0,)), ((), ())))
            Xa = Xa + jnp.dot(Ms[h], oms[h] * Ye)
        Xn = dv * Xa
        if layer == 0:
            X = jnp.maximum(
                jax.lax.dot_general(Xn, W0, (((1,), (1,)), ((), ())))
                + b0_row, 0.0)
        else:
            return (jax.lax.dot_general(Xn, W1, (((1,), (1,)), ((), ())))
                    + b1_row)


def _kern(xc_ref, zc_ref, w_row_ref, bemb_row_ref, WqT_ref, WkT_ref,
          bq_ref, bk_ref, Wg_ref, bg_col_ref, W0_ref, b0_row_ref,
          W1_ref, b1_row_ref, out_ref):
    f32 = jnp.float32
    row_io = jax.lax.broadcasted_iota(jnp.int32, (_G, _G), 0)
    col_io = jax.lax.broadcasted_iota(jnp.int32, (_G, _G), 1)
    eye = (row_io == col_io).astype(f32)
    ones_col = jnp.ones((_G, 1), f32)
    for b in range(_BPS):
        out_ref[b] = _one_batch(
            xc_ref[b], zc_ref[b], w_row_ref[...], bemb_row_ref[...],
            WqT_ref[...], WkT_ref[...], bq_ref[...], bk_ref[...],
            Wg_ref[...], bg_col_ref[...], W0_ref[...], b0_row_ref[...],
            W1_ref[...], b1_row_ref[...], row_io, eye, ones_col)


def kernel(X_gene, Z_effect, W_emb, b_emb, in_proj_w, in_proj_b, W_gate,
           b_gate, W_conv0, b_conv0, W_conv1, b_conv1):
    f32 = jnp.float32
    xc = X_gene.astype(f32).reshape(_B, _G, 1)
    zc = Z_effect.astype(f32).reshape(_B, -1, 1)
    w_row = W_emb[:, 0][None, :].astype(f32)
    bemb_row = b_emb[None, :].astype(f32)
    WqT = in_proj_w[0:_D].T.astype(f32)
    WkT = in_proj_w[_D:2 * _D].T.astype(f32)
    bq = in_proj_b[None, 0:_D].astype(f32)
    bk = in_proj_b[None, _D:2 * _D].astype(f32)
    bg_col = b_gate[:, None].astype(f32)
    b0_row = b_conv0[None, :].astype(f32)
    b1_row = b_conv1[None, :].astype(f32)

    full = lambda arr: pl.BlockSpec(arr.shape, lambda i: (0,) * arr.ndim)
    return pl.pallas_call(
        _kern,
        grid=(_B // _BPS,),
        in_specs=[
            pl.BlockSpec((_BPS, _G, 1), lambda i: (i, 0, 0)),
            pl.BlockSpec((_BPS, zc.shape[1], 1), lambda i: (i, 0, 0)),
            full(w_row), full(bemb_row), full(WqT), full(WkT),
            full(bq), full(bk), full(W_gate), full(bg_col),
            full(W_conv0), full(b0_row), full(W_conv1), full(b1_row),
        ],
        out_specs=pl.BlockSpec((_BPS, _G, _D), lambda i: (i, 0, 0)),
        out_shape=jax.ShapeDtypeStruct((_B, _G, _D), f32),
    )(xc, zc, w_row, bemb_row, WqT, WkT, bq, bk, W_gate.astype(f32),
      bg_col, W_conv0.astype(f32), b0_row, W_conv1.astype(f32), b1_row)


# speedup vs baseline: 1.1961x; 1.1961x over previous
import jax
import jax.numpy as jnp
from jax.experimental import pallas as pl

_B = 128
_G = 256
_D = 32
_H = 4
_HD = _D // _H
_E = _H * _G
_BPS = 8


def _one_batch(x_col, z_col, w_row, bemb_row, WqT, WkT, bq, bk, Wg, bg_col,
               W0, b0_row, W1, b1_row, row_io, eye, ones_col):
    f32 = jnp.float32
    X_emb = x_col * w_row + bemb_row
    q = jnp.dot(X_emb, WqT) + bq
    k = jnp.dot(X_emb, WkT) + bk

    Ms = []
    for h in range(_H):
        qh = q[:, h * _HD:(h + 1) * _HD]
        kh = k[:, h * _HD:(h + 1) * _HD]
        st = jax.lax.dot_general(kh, qh, (((1,), (1,)), ((), ())))
        y = st
        for _ in range(5):
            cur = jnp.max(y, axis=0, keepdims=True)
            idx_of = jnp.where(y == cur, row_io, _G)
            jmin = jnp.min(idx_of, axis=0, keepdims=True)
            y = jnp.where(row_io == jmin, -jnp.inf, y)
        Ms.append(jnp.maximum((y == -jnp.inf).astype(f32), eye))

    oms = []
    Dv = jnp.zeros((_G, 1), f32)
    for h in range(_H):
        g_col = jax.nn.sigmoid(jnp.dot(Wg[h * _G:(h + 1) * _G, :], z_col)
                               + bg_col[h * _G:(h + 1) * _G, :])
        nnz = jax.lax.dot_general(Ms[h], ones_col,
                                  (((0,), (0,)), ((), ())))
        de = 1.0 / (g_col * nnz + 1e-8)
        oms.append(g_col * g_col * de)
        Dv = Dv + jnp.dot(Ms[h], g_col)
    dv = jax.lax.rsqrt(Dv + 1e-8)

    X = X_emb
    for layer in range(2):
        Xs = dv * X
        Xa = jnp.zeros((_G, _D), f32)
        for h in range(_H):
            Ye = jax.lax.dot_general(Ms[h], Xs,
                                     (((0,), (0,)), ((), ())))
            Xa = Xa + jnp.dot(Ms[h], oms[h] * Ye)
        Xn = dv * Xa
        if layer == 0:
            X = jnp.maximum(
                jax.lax.dot_general(Xn, W0, (((1,), (1,)), ((), ())))
                + b0_row, 0.0)
        else:
            return (jax.lax.dot_general(Xn, W1, (((1,), (1,)), ((), ())))
                    + b1_row)


def _kern(xc_ref, zc_ref, w_row_ref, bemb_row_ref, WqT_ref, WkT_ref,
          bq_ref, bk_ref, Wg_ref, bg_col_ref, W0_ref, b0_row_ref,
          W1_ref, b1_row_ref, out_ref):
    f32 = jnp.float32
    row_io = jax.lax.broadcasted_iota(jnp.int32, (_G, _G), 0)
    col_io = jax.lax.broadcasted_iota(jnp.int32, (_G, _G), 1)
    eye = (row_io == col_io).astype(f32)
    ones_col = jnp.ones((_G, 1), f32)
    for b in range(_BPS):
        out_ref[b] = _one_batch(
            xc_ref[b], zc_ref[b], w_row_ref[...], bemb_row_ref[...],
            WqT_ref[...], WkT_ref[...], bq_ref[...], bk_ref[...],
            Wg_ref[...], bg_col_ref[...], W0_ref[...], b0_row_ref[...],
            W1_ref[...], b1_row_ref[...], row_io, eye, ones_col)


def kernel(X_gene, Z_effect, W_emb, b_emb, in_proj_w, in_proj_b, W_gate,
           b_gate, W_conv0, b_conv0, W_conv1, b_conv1):
    f32 = jnp.float32
    xc = X_gene.astype(f32).reshape(_B, _G, 1)
    zc = Z_effect.astype(f32).reshape(_B, -1, 1)
    w_row = W_emb[:, 0][None, :].astype(f32)
    bemb_row = b_emb[None, :].astype(f32)
    WqT = in_proj_w[0:_D].T.astype(f32)
    WkT = in_proj_w[_D:2 * _D].T.astype(f32)
    bq = in_proj_b[None, 0:_D].astype(f32)
    bk = in_proj_b[None, _D:2 * _D].astype(f32)
    bg_col = b_gate[:, None].astype(f32)
    b0_row = b_conv0[None, :].astype(f32)
    b1_row = b_conv1[None, :].astype(f32)

    full = lambda arr: pl.BlockSpec(arr.shape, lambda i: (0,) * arr.ndim)
    return pl.pallas_call(
        _kern,
        grid=(_B // _BPS,),
        in_specs=[
            pl.BlockSpec((_BPS, _G, 1), lambda i: (i, 0, 0)),
            pl.BlockSpec((_BPS, zc.shape[1], 1), lambda i: (i, 0, 0)),
            full(w_row), full(bemb_row), full(WqT), full(WkT),
            full(bq), full(bk), full(W_gate), full(bg_col),
            full(W_conv0), full(b0_row), full(W_conv1), full(b1_row),
        ],
        out_specs=pl.BlockSpec((_BPS, _G, _D), lambda i: (i, 0, 0)),
        out_shape=jax.ShapeDtypeStruct((_B, _G, _D), f32),
    )(xc, zc, w_row, bemb_row, WqT, WkT, bq, bk, W_gate.astype(f32),
      bg_col, W_conv0.astype(f32), b0_row, W_conv1.astype(f32), b1_row)
